# baseline (speedup 1.0000x reference)
import functools
import jax
import jax.numpy as jnp
from jax import lax
from jax.experimental import pallas as pl
from jax.experimental.pallas import tpu as pltpu
from jax.experimental.pallas import tpu_sc as plsc

_IN_SHAPE = (96, 512)
_LO, _HI = 128, 300
_W = _HI - _LO
_EPS_LOG = 0.001
_COLS = 128
_LOG10_E = 0.4342944819032518

_NW = 16
_NB = 2048
_MIN32 = jnp.int32(-2147483648)


def _sc_select_body(k0, chunk, nvec,
                    x_hbm, eps_hbm, data_v, hist_v, allh_v, eps_v, shared):
    wid = lax.axis_index("s")
    base = wid * chunk
    pltpu.sync_copy(x_hbm.at[pl.ds(base, chunk)], data_v)

    zeros16 = jnp.zeros((16,), jnp.int32)
    ones16 = jnp.ones((16,), jnp.int32)
    prefix = jnp.int32(0)
    kk = jnp.int32(k0)
    result = jnp.int32(0)

    for rnd in range(3):
        def zb(i, _):
            hist_v[pl.ds(i * 16, 16)] = zeros16
            return 0
        lax.fori_loop(0, _NB // 16, zb, 0)

        def sb(i, _, rnd=rnd, prefix=prefix):
            xv = data_v[pl.ds(i * 16, 16)]
            b = lax.bitcast_convert_type(xv, jnp.int32)
            flip = jnp.where(b < 0, jnp.int32(-1), _MIN32)
            u = b ^ flip
            if rnd == 0:
                bin_ = lax.shift_right_logical(u, 21)
                plsc.addupdate_scatter(hist_v, [bin_], ones16)
            elif rnd == 1:
                mask = lax.shift_right_logical(u, 21) == prefix
                bin_ = lax.shift_right_logical(u, 10) & jnp.int32(0x7FF)
                plsc.addupdate_scatter(hist_v, [bin_], ones16, mask=mask)
            else:
                mask = lax.shift_right_logical(u, 10) == prefix
                bin_ = u & jnp.int32(0x3FF)
                plsc.addupdate_scatter(hist_v, [bin_], ones16, mask=mask)
            return 0
        lax.fori_loop(0, nvec, sb, 0)

        pltpu.sync_copy(hist_v, shared.at[pl.ds(wid * _NB, _NB)])
        plsc.subcore_barrier()
        pltpu.sync_copy(shared, allh_v)
        plsc.subcore_barrier()

        def cb(i, carry, kk=kk):
            run, nbelow, kb = carry
            acc = allh_v[pl.ds(i * 16, 16)]
            for r in range(1, _NW):
                acc = acc + allh_v[pl.ds(r * _NB + i * 16, 16)]
            cv = jnp.cumsum(acc) + run
            m = cv <= kk
            nbelow = nbelow + jnp.sum(jnp.where(m, jnp.int32(1), jnp.int32(0)))
            kb = kb + jnp.sum(jnp.where(m, acc, jnp.int32(0)))
            run = run + jnp.sum(acc)
            return (run, nbelow, kb)

        _, binidx, kb = lax.fori_loop(
            0, _NB // 16, cb, (jnp.int32(0), jnp.int32(0), jnp.int32(0)))

        if rnd == 0:
            prefix = binidx
        elif rnd == 1:
            prefix = (prefix << 11) | binidx
        else:
            result = (prefix << 10) | binidx
        kk = kk - kb

    vs = result ^ _MIN32
    fb = jnp.where(vs >= 0, vs, vs ^ jnp.int32(0x7FFFFFFF))
    fbv = jnp.full((16,), fb, dtype=jnp.int32)
    eps_v[...] = plsc.bitcast(fbv, jnp.float32)

    @pl.when(wid == 0)
    def _():
        pltpu.sync_copy(eps_v, eps_hbm)


def _make_sc_select(n, k):
    chunk = n // _NW
    nvec = chunk // 16
    mesh = plsc.VectorSubcoreMesh(
        core_axis_name="c", subcore_axis_name="s", num_cores=1)
    return functools.partial(
        pl.kernel,
        mesh=mesh,
        out_type=jax.ShapeDtypeStruct((16,), jnp.float32),
        scratch_types=[
            pltpu.VMEM((chunk,), jnp.float32),
            pltpu.VMEM((_NB,), jnp.int32),
            pltpu.VMEM((_NW * _NB,), jnp.int32),
            pltpu.VMEM((16,), jnp.float32),
            pltpu.MemorySpace.VMEM_SHARED((_NW * _NB,), jnp.int32),
        ],
        compiler_params=pltpu.CompilerParams(needs_layout_passes=False),
    )(functools.partial(_sc_select_body, k, chunk, nvec))


def _tc_body(eps_ref, x_ref, o_ref):
    t = jnp.maximum(eps_ref[0], jnp.float32(_EPS_LOG))
    z = jnp.log(jnp.maximum(x_ref[...], t)) * jnp.float32(_LOG10_E)
    o_ref[...] = z
    zmin = jnp.min(o_ref[...])
    zmax = jnp.max(o_ref[...])
    o_ref[...] = (o_ref[...] - zmin) / (zmax - zmin)


@jax.jit
def kernel(x):
    xb = x.reshape((-1,) + _IN_SHAPE)
    bsz = xb.shape[0]
    n = bsz * _IN_SHAPE[0] * _W
    rows = n // _COLS
    k = int(0.1 * n)
    xs = xb[:, :, _LO:_HI].reshape(rows, _COLS)

    eps = _make_sc_select(n, k)(xs.reshape(-1))[:1]

    out = pl.pallas_call(
        _tc_body,
        out_shape=jax.ShapeDtypeStruct((rows, _COLS), jnp.float32),
        in_specs=[
            pl.BlockSpec(memory_space=pltpu.SMEM),
            pl.BlockSpec(memory_space=pltpu.VMEM),
        ],
        out_specs=pl.BlockSpec(memory_space=pltpu.VMEM),
    )(eps, xs)
    return out.reshape(bsz, _IN_SHAPE[0], _W)


# speedup vs baseline: 1.1200x; 1.1200x over previous
import functools
import jax
import jax.numpy as jnp
from jax import lax
from jax.experimental import pallas as pl
from jax.experimental.pallas import tpu as pltpu
from jax.experimental.pallas import tpu_sc as plsc

_IN_SHAPE = (96, 512)
_LO, _HI = 128, 300
_W = _HI - _LO
_EPS_LOG = 0.001
_COLS = 128
_LOG10_E = 0.4342944819032518

_NW = 16
_NB = 2048
_MIN32 = jnp.int32(-2147483648)


_UN = 4


def _sc_select_body(k0, chunk, nvec,
                    x_hbm, eps_hbm, data_v, hist_v, slice_v, red_v, comb_v,
                    eps_v, sh_hist, sh_comb):
    wid = lax.axis_index("s")
    base = wid * chunk
    pltpu.sync_copy(x_hbm.at[pl.ds(base, chunk)], data_v)

    zeros16 = jnp.zeros((16,), jnp.int32)
    ones16 = jnp.ones((16,), jnp.int32)
    prefix = jnp.int32(0)
    kk = jnp.int32(k0)
    result = jnp.int32(0)
    nsl = _NB // _NW

    for rnd in range(3):
        def zb(i, _):
            hist_v[pl.ds(i * 16, 16)] = zeros16
            return 0
        lax.fori_loop(0, _NB // 16, zb, 0)

        def sb(i, _, rnd=rnd, prefix=prefix):
            for jj in range(_UN):
                off = i * (16 * _UN) + jj * 16
                v = data_v[pl.ds(off, 16)]
                u = lax.bitcast_convert_type(v, jnp.int32)
                if rnd == 0:
                    flip = jnp.where(u < 0, jnp.int32(-1), _MIN32)
                    u = u ^ flip
                    data_v[pl.ds(off, 16)] = lax.bitcast_convert_type(
                        u, jnp.float32)
                    bin_ = lax.shift_right_logical(u, 21)
                    plsc.addupdate_scatter(hist_v, [bin_], ones16)
                elif rnd == 1:
                    mask = lax.shift_right_logical(u, 21) == prefix
                    bin_ = lax.shift_right_logical(u, 10) & jnp.int32(0x7FF)
                    plsc.addupdate_scatter(hist_v, [bin_], ones16, mask=mask)
                else:
                    mask = lax.shift_right_logical(u, 10) == prefix
                    bin_ = u & jnp.int32(0x3FF)
                    plsc.addupdate_scatter(hist_v, [bin_], ones16, mask=mask)
            return 0
        lax.fori_loop(0, nvec // _UN, sb, 0)

        pltpu.sync_copy(hist_v, sh_hist.at[pl.ds(wid * _NB, _NB)])
        plsc.subcore_barrier()
        for r in range(_NW):
            pltpu.sync_copy(sh_hist.at[pl.ds(r * _NB + wid * nsl, nsl)],
                            slice_v.at[pl.ds(r * nsl, nsl)])
        for j in range(nsl // 16):
            acc = slice_v[pl.ds(j * 16, 16)]
            for r in range(1, _NW):
                acc = acc + slice_v[pl.ds(r * nsl + j * 16, 16)]
            red_v[pl.ds(j * 16, 16)] = acc
        pltpu.sync_copy(red_v, sh_comb.at[pl.ds(wid * nsl, nsl)])
        plsc.subcore_barrier()
        pltpu.sync_copy(sh_comb, comb_v)
        plsc.subcore_barrier()

        def cb(i, carry, kk=kk):
            run, nbelow, kb = carry
            acc = comb_v[pl.ds(i * 16, 16)]
            cv = jnp.cumsum(acc) + run
            m = cv <= kk
            nbelow = nbelow + jnp.sum(jnp.where(m, jnp.int32(1), jnp.int32(0)))
            kb = kb + jnp.sum(jnp.where(m, acc, jnp.int32(0)))
            run = run + jnp.sum(acc)
            return (run, nbelow, kb)

        _, binidx, kb = lax.fori_loop(
            0, _NB // 16, cb, (jnp.int32(0), jnp.int32(0), jnp.int32(0)))

        if rnd == 0:
            prefix = binidx
        elif rnd == 1:
            prefix = (prefix << 11) | binidx
        else:
            result = (prefix << 10) | binidx
        kk = kk - kb

    vs = result ^ _MIN32
    fb = jnp.where(vs >= 0, vs, vs ^ jnp.int32(0x7FFFFFFF))
    fbv = jnp.full((16,), fb, dtype=jnp.int32)
    eps_v[...] = plsc.bitcast(fbv, jnp.float32)

    @pl.when(wid == 0)
    def _():
        pltpu.sync_copy(eps_v, eps_hbm)


def _make_sc_select(n, k):
    chunk = n // _NW
    nvec = chunk // 16
    mesh = plsc.VectorSubcoreMesh(
        core_axis_name="c", subcore_axis_name="s", num_cores=1)
    return functools.partial(
        pl.kernel,
        mesh=mesh,
        out_type=jax.ShapeDtypeStruct((16,), jnp.float32),
        scratch_types=[
            pltpu.VMEM((chunk,), jnp.float32),
            pltpu.VMEM((_NB,), jnp.int32),
            pltpu.VMEM((_NB,), jnp.int32),
            pltpu.VMEM((_NB // _NW,), jnp.int32),
            pltpu.VMEM((_NB,), jnp.int32),
            pltpu.VMEM((16,), jnp.float32),
            pltpu.MemorySpace.VMEM_SHARED((_NW * _NB,), jnp.int32),
            pltpu.MemorySpace.VMEM_SHARED((_NB,), jnp.int32),
        ],
        compiler_params=pltpu.CompilerParams(needs_layout_passes=False),
    )(functools.partial(_sc_select_body, k, chunk, nvec))


def _tc_body(eps_ref, x_ref, o_ref):
    t = jnp.maximum(eps_ref[0], jnp.float32(_EPS_LOG))
    z = jnp.log(jnp.maximum(x_ref[...], t)) * jnp.float32(_LOG10_E)
    o_ref[...] = z
    zmin = jnp.min(o_ref[...])
    zmax = jnp.max(o_ref[...])
    o_ref[...] = (o_ref[...] - zmin) / (zmax - zmin)


@jax.jit
def kernel(x):
    xb = x.reshape((-1,) + _IN_SHAPE)
    bsz = xb.shape[0]
    n = bsz * _IN_SHAPE[0] * _W
    rows = n // _COLS
    k = int(0.1 * n)
    xs = xb[:, :, _LO:_HI].reshape(rows, _COLS)

    eps = _make_sc_select(n, k)(xs.reshape(-1))[:1]

    out = pl.pallas_call(
        _tc_body,
        out_shape=jax.ShapeDtypeStruct((rows, _COLS), jnp.float32),
        in_specs=[
            pl.BlockSpec(memory_space=pltpu.SMEM),
            pl.BlockSpec(memory_space=pltpu.VMEM),
        ],
        out_specs=pl.BlockSpec(memory_space=pltpu.VMEM),
    )(eps, xs)
    return out.reshape(bsz, _IN_SHAPE[0], _W)


# speedup vs baseline: 1.1212x; 1.0010x over previous
import functools
import jax
import jax.numpy as jnp
from jax import lax
from jax.experimental import pallas as pl
from jax.experimental.pallas import tpu as pltpu
from jax.experimental.pallas import tpu_sc as plsc

_IN_SHAPE = (96, 512)
_LO, _HI = 128, 300
_W = _HI - _LO
_EPS_LOG = 0.001
_COLS = 128
_LOG10_E = 0.4342944819032518

_NW = 16
_NB = 2048
_MIN32 = jnp.int32(-2147483648)


_UN = 4


def _sc_select_body(k0, chunk, nvec,
                    x_hbm, eps_hbm, data_v, hist_v, slice_v, red_v, comb_v,
                    eps_v, sh_hist, sh_comb):
    wid = lax.axis_index("s")
    base = wid * chunk
    pltpu.sync_copy(x_hbm.at[pl.ds(base, chunk)], data_v)

    zeros16 = jnp.zeros((16,), jnp.int32)
    ones16 = jnp.ones((16,), jnp.int32)
    prefix = jnp.int32(0)
    kk = jnp.int32(k0)
    result = jnp.int32(0)
    nsl = _NB // _NW

    for rnd in range(3):
        def zb(i, _):
            hist_v[pl.ds(i * 16, 16)] = zeros16
            return 0
        lax.fori_loop(0, _NB // 16, zb, 0)

        def sb(i, _, rnd=rnd, prefix=prefix):
            for jj in range(_UN):
                off = i * (16 * _UN) + jj * 16
                v = data_v[pl.ds(off, 16)]
                u = lax.bitcast_convert_type(v, jnp.int32)
                if rnd == 0:
                    flip = jnp.where(u < 0, jnp.int32(-1), _MIN32)
                    u = u ^ flip
                    data_v[pl.ds(off, 16)] = lax.bitcast_convert_type(
                        u, jnp.float32)
                    bin_ = lax.shift_right_logical(u, 21)
                    plsc.addupdate_scatter(hist_v, [bin_], ones16)
                elif rnd == 1:
                    mask = lax.shift_right_logical(u, 21) == prefix
                    bin_ = lax.shift_right_logical(u, 10) & jnp.int32(0x7FF)
                    plsc.addupdate_scatter(hist_v, [bin_], ones16, mask=mask)
                else:
                    mask = lax.shift_right_logical(u, 10) == prefix
                    bin_ = u & jnp.int32(0x3FF)
                    plsc.addupdate_scatter(hist_v, [bin_], ones16, mask=mask)
            return 0
        lax.fori_loop(0, nvec // _UN, sb, 0)

        pltpu.sync_copy(hist_v, sh_hist.at[pl.ds(wid * _NB, _NB)])
        plsc.subcore_barrier()
        for r in range(_NW):
            pltpu.sync_copy(sh_hist.at[pl.ds(r * _NB + wid * nsl, nsl)],
                            slice_v.at[pl.ds(r * nsl, nsl)])
        for j in range(nsl // 16):
            acc = slice_v[pl.ds(j * 16, 16)]
            for r in range(1, _NW):
                acc = acc + slice_v[pl.ds(r * nsl + j * 16, 16)]
            red_v[pl.ds(j * 16, 16)] = acc
        pltpu.sync_copy(red_v, sh_comb.at[pl.ds(wid * nsl, nsl)])
        plsc.subcore_barrier()
        pltpu.sync_copy(sh_comb, comb_v)
        plsc.subcore_barrier()

        def cb(i, carry, kk=kk):
            run, nbelow, kb = carry
            acc = comb_v[pl.ds(i * 16, 16)]
            cv = jnp.cumsum(acc) + run
            m = cv <= kk
            nbelow = nbelow + jnp.sum(jnp.where(m, jnp.int32(1), jnp.int32(0)))
            kb = kb + jnp.sum(jnp.where(m, acc, jnp.int32(0)))
            run = run + jnp.sum(acc)
            return (run, nbelow, kb)

        _, binidx, kb = lax.fori_loop(
            0, _NB // 16, cb, (jnp.int32(0), jnp.int32(0), jnp.int32(0)))

        if rnd == 0:
            prefix = binidx
        elif rnd == 1:
            prefix = (prefix << 11) | binidx
        else:
            result = (prefix << 10) | binidx
        kk = kk - kb

    vs = result ^ _MIN32
    fb = jnp.where(vs >= 0, vs, vs ^ jnp.int32(0x7FFFFFFF))
    fbv = jnp.full((16,), fb, dtype=jnp.int32)
    eps_v[...] = plsc.bitcast(fbv, jnp.float32)

    @pl.when(wid == 0)
    def _():
        pltpu.sync_copy(eps_v, eps_hbm)


def _make_sc_select(n, k):
    chunk = n // _NW
    nvec = chunk // 16
    mesh = plsc.VectorSubcoreMesh(
        core_axis_name="c", subcore_axis_name="s", num_cores=1)
    return functools.partial(
        pl.kernel,
        mesh=mesh,
        out_type=jax.ShapeDtypeStruct((16,), jnp.float32),
        scratch_types=[
            pltpu.VMEM((chunk,), jnp.float32),
            pltpu.VMEM((_NB,), jnp.int32),
            pltpu.VMEM((_NB,), jnp.int32),
            pltpu.VMEM((_NB // _NW,), jnp.int32),
            pltpu.VMEM((_NB,), jnp.int32),
            pltpu.VMEM((16,), jnp.float32),
            pltpu.MemorySpace.VMEM_SHARED((_NW * _NB,), jnp.int32),
            pltpu.MemorySpace.VMEM_SHARED((_NB,), jnp.int32),
        ],
        compiler_params=pltpu.CompilerParams(needs_layout_passes=False),
    )(functools.partial(_sc_select_body, k, chunk, nvec))


def _tc_log_body(x_ref, z_ref):
    z_ref[...] = jnp.log(
        jnp.maximum(x_ref[...], jnp.float32(_EPS_LOG))
    ) * jnp.float32(_LOG10_E)


def _tc_norm_body(eps_ref, z_ref, o_ref):
    t = jnp.maximum(eps_ref[0], jnp.float32(_EPS_LOG))
    ltv = jnp.log(jnp.full((8, _COLS), t)) * jnp.float32(_LOG10_E)
    lt = ltv[0, 0]
    z = z_ref[...]
    zmax = jnp.maximum(jnp.max(z), lt)
    o_ref[...] = (jnp.maximum(z, lt) - lt) / (zmax - lt)


@jax.jit
def kernel(x):
    xb = x.reshape((-1,) + _IN_SHAPE)
    bsz = xb.shape[0]
    n = bsz * _IN_SHAPE[0] * _W
    rows = n // _COLS
    k = int(0.1 * n)
    xs = xb[:, :, _LO:_HI].reshape(rows, _COLS)

    eps = _make_sc_select(n, k)(xs.reshape(-1))[:1]

    z = pl.pallas_call(
        _tc_log_body,
        out_shape=jax.ShapeDtypeStruct((rows, _COLS), jnp.float32),
    )(xs)

    out = pl.pallas_call(
        _tc_norm_body,
        out_shape=jax.ShapeDtypeStruct((rows, _COLS), jnp.float32),
        in_specs=[
            pl.BlockSpec(memory_space=pltpu.SMEM),
            pl.BlockSpec(memory_space=pltpu.VMEM),
        ],
        out_specs=pl.BlockSpec(memory_space=pltpu.VMEM),
    )(eps, z)
    return out.reshape(bsz, _IN_SHAPE[0], _W)


# speedup vs baseline: 1.2897x; 1.1503x over previous
import functools
import jax
import jax.numpy as jnp
from jax import lax
from jax.experimental import pallas as pl
from jax.experimental.pallas import tpu as pltpu
from jax.experimental.pallas import tpu_sc as plsc

_IN_SHAPE = (96, 512)
_LO, _HI = 128, 300
_W = _HI - _LO
_EPS_LOG = 0.001
_COLS = 128
_LOG10_E = 0.4342944819032518

_NW = 16
_NB = 2048
_MIN32 = jnp.int32(-2147483648)


_UN = 4


def _sc_select_body(k0, chunk, nvec,
                    x_hbm, eps_hbm, data_v, hist_v, slice_v, red_v, comb_v,
                    eps_v, sh_hist, sh_comb):
    wid = lax.axis_index("s")
    base = wid * chunk
    pltpu.sync_copy(x_hbm.at[pl.ds(base, chunk)], data_v.at[pl.ds(0, chunk)])

    zeros16 = jnp.zeros((16,), jnp.int32)
    ones16 = jnp.ones((16,), jnp.int32)
    prefix = jnp.int32(0)
    kk = jnp.int32(k0)
    result = jnp.int32(0)
    nsl = _NB // _NW

    for rnd in range(3):
        def zb(i, _):
            hist_v[pl.ds(i * 16, 16)] = zeros16
            return 0
        lax.fori_loop(0, _NB // 16, zb, 0)

        if rnd == 0:
            def sb(i, _):
                for jj in range(_UN):
                    off = i * (16 * _UN) + jj * 16
                    v = data_v[pl.ds(off, 16)]
                    u = lax.bitcast_convert_type(v, jnp.int32)
                    flip = jnp.where(u < 0, jnp.int32(-1), _MIN32)
                    u = u ^ flip
                    data_v[pl.ds(off, 16)] = lax.bitcast_convert_type(
                        u, jnp.float32)
                    bin_ = lax.shift_right_logical(u, 21)
                    plsc.addupdate_scatter(hist_v, [bin_], ones16)
                return 0
            lax.fori_loop(0, nvec // _UN, sb, 0)
        elif rnd == 1:
            def sb(i, coff, prefix=prefix):
                for jj in range(_UN):
                    off = i * (16 * _UN) + jj * 16
                    v = data_v[pl.ds(off, 16)]
                    u = lax.bitcast_convert_type(v, jnp.int32)
                    mask = lax.shift_right_logical(u, 21) == prefix
                    bin_ = lax.shift_right_logical(u, 10) & jnp.int32(0x7FF)
                    plsc.addupdate_scatter(hist_v, [bin_], ones16, mask=mask)
                    plsc.store_compressed(
                        data_v.at[pl.ds(coff, 16)], v, mask=mask)
                    coff = coff + jnp.sum(
                        jnp.where(mask, jnp.int32(1), jnp.int32(0)))
                return coff
            ncand = lax.fori_loop(0, nvec // _UN, sb, jnp.int32(0))
            data_v[pl.ds(ncand, 16)] = lax.bitcast_convert_type(
                jnp.full((16,), jnp.int32(-1)), jnp.float32)
        else:
            def sb(i, _, prefix=prefix):
                v = data_v[pl.ds(i * 16, 16)]
                u = lax.bitcast_convert_type(v, jnp.int32)
                mask = lax.shift_right_logical(u, 10) == prefix
                bin_ = u & jnp.int32(0x3FF)
                plsc.addupdate_scatter(hist_v, [bin_], ones16, mask=mask)
                return 0
            nvec2 = lax.shift_right_logical(ncand + jnp.int32(15), 4)
            lax.fori_loop(0, nvec2, sb, 0)

        pltpu.sync_copy(hist_v, sh_hist.at[pl.ds(wid * _NB, _NB)])
        plsc.subcore_barrier()
        for r in range(_NW):
            pltpu.sync_copy(sh_hist.at[pl.ds(r * _NB + wid * nsl, nsl)],
                            slice_v.at[pl.ds(r * nsl, nsl)])
        for j in range(nsl // 16):
            acc = slice_v[pl.ds(j * 16, 16)]
            for r in range(1, _NW):
                acc = acc + slice_v[pl.ds(r * nsl + j * 16, 16)]
            red_v[pl.ds(j * 16, 16)] = acc
        pltpu.sync_copy(red_v, sh_comb.at[pl.ds(wid * nsl, nsl)])
        plsc.subcore_barrier()
        pltpu.sync_copy(sh_comb, comb_v)
        plsc.subcore_barrier()

        def cb(i, carry, kk=kk):
            run, nbelow, kb = carry
            acc = comb_v[pl.ds(i * 16, 16)]
            cv = jnp.cumsum(acc) + run
            m = cv <= kk
            nbelow = nbelow + jnp.sum(jnp.where(m, jnp.int32(1), jnp.int32(0)))
            kb = kb + jnp.sum(jnp.where(m, acc, jnp.int32(0)))
            run = run + jnp.sum(acc)
            return (run, nbelow, kb)

        _, binidx, kb = lax.fori_loop(
            0, _NB // 16, cb, (jnp.int32(0), jnp.int32(0), jnp.int32(0)))

        if rnd == 0:
            prefix = binidx
        elif rnd == 1:
            prefix = (prefix << 11) | binidx
        else:
            result = (prefix << 10) | binidx
        kk = kk - kb

    vs = result ^ _MIN32
    fb = jnp.where(vs >= 0, vs, vs ^ jnp.int32(0x7FFFFFFF))
    fbv = jnp.full((16,), fb, dtype=jnp.int32)
    eps_v[...] = plsc.bitcast(fbv, jnp.float32)

    @pl.when(wid == 0)
    def _():
        pltpu.sync_copy(eps_v, eps_hbm)


def _make_sc_select(n, k):
    chunk = n // _NW
    nvec = chunk // 16
    mesh = plsc.VectorSubcoreMesh(
        core_axis_name="c", subcore_axis_name="s", num_cores=1)
    return functools.partial(
        pl.kernel,
        mesh=mesh,
        out_type=jax.ShapeDtypeStruct((16,), jnp.float32),
        scratch_types=[
            pltpu.VMEM((chunk + 16,), jnp.float32),
            pltpu.VMEM((_NB,), jnp.int32),
            pltpu.VMEM((_NB,), jnp.int32),
            pltpu.VMEM((_NB // _NW,), jnp.int32),
            pltpu.VMEM((_NB,), jnp.int32),
            pltpu.VMEM((16,), jnp.float32),
            pltpu.MemorySpace.VMEM_SHARED((_NW * _NB,), jnp.int32),
            pltpu.MemorySpace.VMEM_SHARED((_NB,), jnp.int32),
        ],
        compiler_params=pltpu.CompilerParams(needs_layout_passes=False),
    )(functools.partial(_sc_select_body, k, chunk, nvec))


def _tc_body(eps_ref, x_ref, o_ref):
    t = jnp.maximum(eps_ref[0], jnp.float32(_EPS_LOG))
    ltv = jnp.log(jnp.full((8, _COLS), t)) * jnp.float32(_LOG10_E)
    lt = ltv[0, 0]
    o_ref[...] = jnp.log(
        jnp.maximum(x_ref[...], t)) * jnp.float32(_LOG10_E)
    zmax = jnp.max(o_ref[...])
    s = jnp.float32(1.0) / (zmax - lt)
    o_ref[...] = (o_ref[...] - lt) * s


@jax.jit
def kernel(x):
    xb = x.reshape((-1,) + _IN_SHAPE)
    bsz = xb.shape[0]
    n = bsz * _IN_SHAPE[0] * _W
    rows = n // _COLS
    k = int(0.1 * n)
    xs = xb[:, :, _LO:_HI].reshape(rows, _COLS)

    eps = _make_sc_select(n, k)(xs.reshape(-1))[:1]

    out = pl.pallas_call(
        _tc_body,
        out_shape=jax.ShapeDtypeStruct((rows, _COLS), jnp.float32),
        in_specs=[
            pl.BlockSpec(memory_space=pltpu.SMEM),
            pl.BlockSpec(memory_space=pltpu.VMEM),
        ],
        out_specs=pl.BlockSpec(memory_space=pltpu.VMEM),
    )(eps, xs)
    return out.reshape(bsz, _IN_SHAPE[0], _W)
